# Initial kernel scaffold; baseline (speedup 1.0000x reference)
import jax
import jax.numpy as jnp
from jax.experimental import pallas as pl


def kernel(feat, edge_index, weight, bias):
    raise NotImplementedError("write your pallas kernel here")


# speedup vs baseline: 8.1560x; 8.1560x over previous
import functools

import jax
import jax.numpy as jnp
from jax import lax
from jax.experimental import pallas as pl
from jax.experimental.pallas import tpu as pltpu
from jax.experimental.pallas import tpu_sc as plsc

NC, NS, L = 2, 16, 16
NW = NC * NS
N_NODES = 10000
NP = 10240
D = 128
N_EDGES = 320000
EPW = 10240
NPE = NW * EPW
CH = 128
NCH = EPW // CH
NCQ = NCH // 8
RPT = NP // NS

_MESH = plsc.VectorSubcoreMesh(core_axis_name="c", subcore_axis_name="s")


def _deg_body(src_hbm, dst_hbm, ones_hbm, z1_hbm,
              outdeg_hbm, indeg_hbm,
              src_v, dst_v, ones_v, z1_v,
              outdeg_s, indeg_s):
    c = lax.axis_index("c")
    s = lax.axis_index("s")
    w = c * NS + s
    pltpu.sync_copy(z1_hbm, z1_v)
    pltpu.sync_copy(z1_v, outdeg_s.at[pl.ds(s * RPT, RPT)])
    pltpu.sync_copy(z1_v, indeg_s.at[pl.ds(s * RPT, RPT)])
    pltpu.sync_copy(ones_hbm, ones_v)
    pltpu.sync_copy(src_hbm.at[w], src_v)
    pltpu.sync_copy(dst_hbm.at[w], dst_v)
    plsc.subcore_barrier()

    def step(j, carry):
        q = j // 8
        r = j % 8
        pltpu.sync_copy(ones_v, outdeg_s.at[src_v.at[q, r]], add=True)
        pltpu.sync_copy(ones_v, indeg_s.at[dst_v.at[q, r]], add=True)
        return carry

    lax.fori_loop(0, NCH, step, 0)
    plsc.subcore_barrier()
    pltpu.sync_copy(outdeg_s.at[pl.ds(s * RPT, RPT)], z1_v)
    pltpu.sync_copy(z1_v, outdeg_hbm.at[c, pl.ds(s * RPT, RPT)])
    pltpu.sync_copy(indeg_s.at[pl.ds(s * RPT, RPT)], z1_v)
    pltpu.sync_copy(z1_v, indeg_hbm.at[c, pl.ds(s * RPT, RPT)])


_deg_kernel = functools.partial(
    pl.kernel,
    out_type=(
        jax.ShapeDtypeStruct((NC, NP), jnp.float32),
        jax.ShapeDtypeStruct((NC, NP), jnp.float32),
    ),
    mesh=_MESH,
    scratch_types=[
        pltpu.VMEM((NCQ, 8, CH), jnp.int32),
        pltpu.VMEM((NCQ, 8, CH), jnp.int32),
        pltpu.VMEM((CH,), jnp.float32),
        pltpu.VMEM((RPT,), jnp.float32),
        pltpu.VMEM_SHARED((NP,), jnp.float32),
        pltpu.VMEM_SHARED((NP,), jnp.float32),
    ],
)(_deg_body)


def _agg_body(src_hbm, dst_hbm, feat_hbm, z_hbm,
              agg_hbm,
              src_v, dst_v, rows_v, sem,
              acc_s):
    c = lax.axis_index("c")
    s = lax.axis_index("s")
    w = c * NS + s
    pltpu.sync_copy(z_hbm, rows_v)
    for k in range(RPT // CH):
        pltpu.sync_copy(rows_v, acc_s.at[pl.ds(s * RPT + k * CH, CH)])
    pltpu.sync_copy(src_hbm.at[w], src_v)
    pltpu.sync_copy(dst_hbm.at[w], dst_v)
    plsc.subcore_barrier()

    def step(j, carry):
        q = j // 8
        r = j % 8
        pltpu.async_copy(feat_hbm.at[src_v.at[q, r]], rows_v, sem).wait()
        pltpu.sync_copy(rows_v, acc_s.at[dst_v.at[q, r]], add=True)
        return carry

    lax.fori_loop(0, NCH, step, 0)
    plsc.subcore_barrier()
    for k in range(RPT // CH):
        pltpu.sync_copy(acc_s.at[pl.ds(s * RPT + k * CH, CH)], rows_v)
        pltpu.sync_copy(rows_v, agg_hbm.at[c, pl.ds(s * RPT + k * CH, CH)])


_agg_kernel = functools.partial(
    pl.kernel,
    out_type=jax.ShapeDtypeStruct((NC, NP, D), jnp.float32),
    mesh=_MESH,
    scratch_types=[
        pltpu.VMEM((NCQ, 8, CH), jnp.int32),
        pltpu.VMEM((NCQ, 8, CH), jnp.int32),
        pltpu.VMEM((CH, D), jnp.float32),
        pltpu.SemaphoreType.DMA,
        pltpu.VMEM_SHARED((NP, D), jnp.float32),
    ],
)(_agg_body)


R2 = 512
NB = NP // R2


def _scale_body(feat_ref, od_ref, fs_ref):
    od = od_ref[0] + od_ref[1]
    nl = lax.rsqrt(jnp.maximum(od, 1.0))
    fs_ref[...] = feat_ref[...] * nl


def _scale_call(feat_p, outdeg):
    return pl.pallas_call(
        _scale_body,
        grid=(NB,),
        in_specs=[
            pl.BlockSpec((R2, D), lambda b: (b, 0)),
            pl.BlockSpec((NC, R2, 1), lambda b: (0, b, 0)),
        ],
        out_specs=pl.BlockSpec((R2, D), lambda b: (b, 0)),
        out_shape=jax.ShapeDtypeStruct((NP, D), jnp.float32),
    )(feat_p, outdeg)


def _mm_body(agg_ref, id_ref, w_ref, b_ref, out_ref):
    a = agg_ref[0] + agg_ref[1]
    idg = id_ref[0] + id_ref[1]
    nr = lax.rsqrt(jnp.maximum(idg, 1.0))
    mm = jnp.dot(a, w_ref[...], preferred_element_type=jnp.float32)
    out_ref[...] = mm * nr + b_ref[...]


def _mm_call(agg, indeg, weight, bias2d):
    return pl.pallas_call(
        _mm_body,
        grid=(NB,),
        in_specs=[
            pl.BlockSpec((NC, R2, D), lambda b: (0, b, 0)),
            pl.BlockSpec((NC, R2, 1), lambda b: (0, b, 0)),
            pl.BlockSpec((D, D), lambda b: (0, 0)),
            pl.BlockSpec((1, D), lambda b: (0, 0)),
        ],
        out_specs=pl.BlockSpec((R2, D), lambda b: (b, 0)),
        out_shape=jax.ShapeDtypeStruct((NP, D), jnp.float32),
    )(agg, indeg, weight, bias2d)


def kernel(feat, edge_index, weight, bias):
    feat = feat.astype(jnp.float32)
    ei = edge_index.astype(jnp.int32)
    npad = NPE - N_EDGES
    padv = N_NODES + (jnp.arange(npad, dtype=jnp.int32) % L)
    src_p = jnp.concatenate([ei[0], padv]).reshape(NW, NCQ, 8, CH)
    dst_p = jnp.concatenate([ei[1], padv]).reshape(NW, NCQ, 8, CH)
    feat_p = jnp.pad(feat, ((0, NP - N_NODES), (0, 0)))

    ones1 = jnp.ones((CH,), jnp.float32)
    z1 = jnp.zeros((RPT,), jnp.float32)
    zrow = jnp.zeros((CH, D), jnp.float32)

    outdeg, indeg = _deg_kernel(src_p, dst_p, ones1, z1)
    feat_s = _scale_call(feat_p, outdeg.reshape(NC, NP, 1))
    agg = _agg_kernel(src_p, dst_p, feat_s, zrow)
    out = _mm_call(agg, indeg.reshape(NC, NP, 1), weight, bias.reshape(1, D))
    return out[:N_NODES]
